# Initial kernel scaffold; baseline (speedup 1.0000x reference)
import jax
import jax.numpy as jnp
from jax.experimental import pallas as pl


def kernel(node_features, edges, params):
    raise NotImplementedError("write your pallas kernel here")


# speedup vs baseline: 10.4929x; 10.4929x over previous
import functools

import jax
import jax.numpy as jnp
from jax import lax
from jax.experimental import pallas as pl
from jax.experimental.pallas import tpu as pltpu
from jax.experimental.pallas import tpu_sc as plsc

_N = 10000
_NP = 10240
_RB = 1280
_NRB = _NP // _RB
_NT = 16
_SL = 128
_NSA = 160
_NSB = 80
_EP = _NT * _NSA * _SL
_CC = 128
_F32 = jnp.float32
_HIGH = lax.Precision.HIGHEST


def _layout(c):
    return max(1, c // _CC), _CC


def _edge_loop(eil_hbm, w, nsh, tbl, acc, iv0, iv1, buf0, buf1,
               sem0, sem1, semi, sems):
    def do_pair(ivc, ivn, pn):
        ci = pltpu.async_copy(eil_hbm.at[w, pl.ds(2 * pn, 2)], ivn, semi)
        c0 = pltpu.async_copy(tbl.at[ivc.at[0, 0]], buf0, sem0)
        c1 = pltpu.async_copy(tbl.at[ivc.at[1, 0]], buf1, sem1)
        c0.wait()
        s0 = pltpu.async_copy(buf0, acc.at[ivc.at[0, 1]], sems, add=True)
        c1.wait()
        s1 = pltpu.async_copy(buf1, acc.at[ivc.at[1, 1]], sems, add=True)
        s0.wait()
        s1.wait()
        ci.wait()

    def step(q, carry):
        p0 = 2 * q
        do_pair(iv0, iv1, p0 + 1)
        do_pair(iv1, iv0, jnp.minimum(p0 + 2, nsh - 1))
        return carry

    pltpu.async_copy(eil_hbm.at[w, pl.ds(0, 2)], iv0, semi).wait()
    lax.fori_loop(0, nsh // 2, step, 0)


def _scratch():
    return [
        pltpu.VMEM((2, 2, _SL), jnp.int32),
        pltpu.VMEM((2, 2, _SL), jnp.int32),
        pltpu.VMEM((_SL, _CC), _F32),
        pltpu.VMEM((_SL, _CC), _F32),
        pltpu.VMEM_SHARED((_NP, _CC), _F32),
        pltpu.SemaphoreType.DMA,
        pltpu.SemaphoreType.DMA,
        pltpu.SemaphoreType.DMA,
        pltpu.SemaphoreType.DMA,
    ]


@functools.cache
def _make_agg_shared(nc):
    rows_pt = _NP // _NT
    nsh = _NSA // 2
    mesh = plsc.VectorSubcoreMesh(core_axis_name="c", subcore_axis_name="s")

    def body(y_hbm, eil_hbm, out_hbm,
             iv0, iv1, buf0, buf1, acc, sem0, sem1, semi, sems):
        cid = lax.axis_index("c")
        sid = lax.axis_index("s")
        rsl = pl.ds(sid * rows_pt, rows_pt)
        for ch in range(nc):
            @pl.when(cid == (ch % 2))
            def _():
                tbl = y_hbm.at[ch]
                pltpu.sync_copy(tbl.at[rsl], acc.at[rsl])
                plsc.subcore_barrier()
                _edge_loop(eil_hbm, sid, nsh, tbl, acc,
                           iv0, iv1, buf0, buf1, sem0, sem1, semi, sems)
                plsc.subcore_barrier()
                pltpu.sync_copy(acc.at[rsl], out_hbm.at[ch, rsl])
                plsc.subcore_barrier()

    return pl.kernel(
        body,
        out_type=jax.ShapeDtypeStruct((nc, _NP, _CC), _F32),
        mesh=mesh,
        scratch_types=_scratch(),
    )


@functools.cache
def _make_agg_split():
    rows_pt = _NP // _NT
    nsh = _NSB // 2
    mesh = plsc.VectorSubcoreMesh(core_axis_name="c", subcore_axis_name="s")

    def body(y_hbm, z_hbm, eil_hbm, out_hbm,
             iv0, iv1, buf0, buf1, acc, sem0, sem1, semi, sems):
        cid = lax.axis_index("c")
        sid = lax.axis_index("s")
        wid = cid * _NT + sid
        tbl = y_hbm.at[0]
        rsl = pl.ds(sid * rows_pt, rows_pt)

        @pl.when(cid == 0)
        def _():
            pltpu.sync_copy(tbl.at[rsl], acc.at[rsl])

        @pl.when(cid == 1)
        def _():
            pltpu.sync_copy(z_hbm.at[rsl], acc.at[rsl])

        plsc.subcore_barrier()
        _edge_loop(eil_hbm, wid, nsh, tbl, acc,
                   iv0, iv1, buf0, buf1, sem0, sem1, semi, sems)
        plsc.subcore_barrier()
        pltpu.sync_copy(acc.at[rsl], out_hbm.at[cid, rsl])
        plsc.subcore_barrier()

    return pl.kernel(
        body,
        out_type=jax.ShapeDtypeStruct((2, _NP, _CC), _F32),
        mesh=mesh,
        scratch_types=_scratch(),
    )


def _colnorm(nf_pad):
    def body(x_ref, o_ref, st_ref):
        p = pl.program_id(0)
        rb = pl.program_id(1)

        @pl.when((p == 0) & (rb == 0))
        def _():
            st_ref[...] = jnp.zeros_like(st_ref)

        x = x_ref[...]

        @pl.when(p == 0)
        def _():
            st_ref[...] += (x * x).reshape(_RB // 8, 8, 128).sum(axis=0)

        @pl.when(p == 1)
        def _():
            ss = st_ref[...].sum(axis=0, keepdims=True)
            norm = jnp.maximum(jnp.sqrt(ss), 1e-12)
            o_ref[0] = x / norm

    return pl.pallas_call(
        body,
        grid=(2, _NRB),
        in_specs=[pl.BlockSpec((_RB, 128), lambda p, rb: (rb, 0))],
        out_specs=pl.BlockSpec((1, _RB, 128), lambda p, rb: (0, rb, 0)),
        out_shape=jax.ShapeDtypeStruct((1, _NP, 128), _F32),
        scratch_shapes=[pltpu.VMEM((8, 128), _F32)],
    )(nf_pad)


def _dinv_kernel(deg):
    def body(d_ref, o_ref):
        d = d_ref[0, :, 0:1] + d_ref[1, :, 0:1]
        dv = jnp.where(d > 0, lax.rsqrt(d), 0.0)
        o_ref[...] = jnp.broadcast_to(dv, (_RB, 128))

    return pl.pallas_call(
        body,
        grid=(_NRB,),
        in_specs=[pl.BlockSpec((2, _RB, 128), lambda rb: (0, rb, 0))],
        out_specs=pl.BlockSpec((_RB, 128), lambda rb: (rb, 0)),
        out_shape=jax.ShapeDtypeStruct((_NP, 128), _F32),
    )(deg)


def _mm(act, dinv, w):
    nci, ci = act.shape[0], act.shape[2]
    cin, cout = w.shape
    nco, cco = _layout(cout)
    w3 = w.astype(_F32).reshape(cin, nco, cco).transpose(1, 0, 2)

    def body(a_ref, d_ref, w_ref, o_ref):
        d = d_ref[...][:, 0:1]
        acc = jnp.zeros((_RB, cco), _F32)
        for i in range(nci):
            a = a_ref[i] * d
            acc = acc + lax.dot_general(
                a, w_ref[0, i * ci:(i + 1) * ci, :],
                (((1,), (0,)), ((), ())),
                precision=_HIGH, preferred_element_type=_F32)
        o_ref[0] = acc

    return pl.pallas_call(
        body,
        grid=(_NRB, nco),
        in_specs=[
            pl.BlockSpec((nci, _RB, ci), lambda rb, co: (0, rb, 0)),
            pl.BlockSpec((_RB, 128), lambda rb, co: (rb, 0)),
            pl.BlockSpec((1, cin, cco), lambda rb, co: (co, 0, 0)),
        ],
        out_specs=pl.BlockSpec((1, _RB, cco), lambda rb, co: (co, rb, 0)),
        out_shape=jax.ShapeDtypeStruct((nco, _NP, cco), _F32),
    )(act, dinv, w3)


def _post(u, dinv, pvec, mode, extra=None):
    c = pvec.shape[1]
    nc = c // _CC
    partial = nc == 1
    nu = u.shape[0]
    ne = extra.shape[0] if extra is not None else 0
    has_extra = mode in ("res", "down")

    def u_row(ref, i):
        return ref[0] + ref[1] if partial else ref[i]

    def body(*refs):
        if has_extra:
            u_ref, d_ref, p_ref, e_ref, o_ref, st_ref = refs
        else:
            u_ref, d_ref, p_ref, o_ref, st_ref = refs
        p = pl.program_id(0)
        rb = pl.program_id(1)

        @pl.when((p == 0) & (rb == 0))
        def _():
            st_ref[...] = jnp.zeros_like(st_ref)

        d = d_ref[...][:, 0:1]
        rows = rb * _RB + lax.broadcasted_iota(jnp.int32, (_RB, 1), 0)
        msk = rows < _N

        @pl.when(p == 0)
        def _():
            for i in range(nc):
                cs = pl.ds(i * _CC, _CC)
                xi = u_row(u_ref, i) * d + p_ref[0, cs][None, :]
                xm = jnp.where(msk, xi, 0.0)
                st_ref[0, :, cs] += xm.reshape(_RB // 8, 8, _CC).sum(axis=0)
                st_ref[1, :, cs] += (xm * xm).reshape(_RB // 8, 8, _CC).sum(axis=0)
                if mode == "down":
                    yi = u_row(e_ref, i) * d + p_ref[3, cs][None, :]
                    ym = jnp.where(msk, yi, 0.0)
                    st_ref[2, :, cs] += ym.reshape(_RB // 8, 8, _CC).sum(axis=0)
                    st_ref[3, :, cs] += (ym * ym).reshape(_RB // 8, 8, _CC).sum(axis=0)

        @pl.when(p == 1)
        def _():
            for i in range(nc):
                cs = pl.ds(i * _CC, _CC)
                xi = u_row(u_ref, i) * d + p_ref[0, cs][None, :]
                if mode != "plain":
                    m = st_ref[0, :, cs].sum(axis=0) / _N
                    v = st_ref[1, :, cs].sum(axis=0) / _N - m * m
                    xi = (p_ref[1, cs][None, :] * (xi - m[None, :])
                          * lax.rsqrt(v + 1e-5)[None, :] + p_ref[2, cs][None, :])
                xi = jnp.maximum(xi, 0.0)
                if mode == "res":
                    xi = xi + e_ref[i]
                if mode == "down":
                    yi = u_row(e_ref, i) * d + p_ref[3, cs][None, :]
                    md = st_ref[2, :, cs].sum(axis=0) / _N
                    vd = st_ref[3, :, cs].sum(axis=0) / _N - md * md
                    yi = (p_ref[4, cs][None, :] * (yi - md[None, :])
                          * lax.rsqrt(vd + 1e-5)[None, :] + p_ref[5, cs][None, :])
                    xi = xi + yi
                o_ref[i] = xi

    in_specs = [
        pl.BlockSpec((nu, _RB, _CC), lambda p, rb: (0, rb, 0)),
        pl.BlockSpec((_RB, 128), lambda p, rb: (rb, 0)),
        pl.BlockSpec((8, c), lambda p, rb: (0, 0)),
    ]
    args = [u, dinv, pvec]
    if has_extra:
        in_specs.append(pl.BlockSpec((ne, _RB, _CC), lambda p, rb: (0, rb, 0)))
        args.append(extra)

    return pl.pallas_call(
        body,
        grid=(2, _NRB),
        in_specs=in_specs,
        out_specs=pl.BlockSpec((nc, _RB, _CC), lambda p, rb: (0, rb, 0)),
        out_shape=jax.ShapeDtypeStruct((nc, _NP, _CC), _F32),
        scratch_shapes=[pltpu.VMEM((4, 8, c), _F32)],
    )(*args)


def _final(u, dinv, pvec):
    def body(u_ref, d_ref, p_ref, o_ref, st_ref):
        rb = pl.program_id(0)

        @pl.when(rb == 0)
        def _():
            st_ref[...] = jnp.zeros_like(st_ref)

        d = d_ref[...][:, 0:1]
        rows = rb * _RB + lax.broadcasted_iota(jnp.int32, (_RB, 1), 0)
        msk = rows < _N
        for i in range(4):
            cs = pl.ds(i * 128, 128)
            xi = u_ref[i] * d + p_ref[0, cs][None, :]
            xm = jnp.where(msk, xi, 0.0)
            st_ref[:, cs] += xm.reshape(_RB // 8, 8, 128).sum(axis=0)

        @pl.when(rb == _NRB - 1)
        def _():
            tot = st_ref[...].sum(axis=0) / _N
            o_ref[...] = jnp.broadcast_to(tot[None, :], (8, 512))

    return pl.pallas_call(
        body,
        grid=(_NRB,),
        in_specs=[
            pl.BlockSpec((4, _RB, 128), lambda rb: (0, rb, 0)),
            pl.BlockSpec((_RB, 128), lambda rb: (rb, 0)),
            pl.BlockSpec((8, 512), lambda rb: (0, 0)),
        ],
        out_specs=pl.BlockSpec((8, 512), lambda rb: (0, 0)),
        out_shape=jax.ShapeDtypeStruct((8, 512), _F32),
        scratch_shapes=[pltpu.VMEM((8, 512), _F32)],
    )(u, dinv, pvec)


def _cpad(c):
    return max(_CC, c)


def _pack_params(c, **rows):
    cp = _cpad(c)
    order = ("b", "g", "bb", "bd", "gd", "bbd")
    out = []
    for k in order:
        v = rows.get(k, None)
        v = jnp.zeros((cp,), _F32) if v is None else jnp.pad(
            v.astype(_F32), (0, cp - c))
        out.append(v)
    out += [jnp.zeros((cp,), _F32)] * 2
    return jnp.stack(out)


def _wpad(w):
    cin, cout = w.shape
    return jnp.pad(w.astype(_F32),
                   ((0, _cpad(cin) - cin), (0, _cpad(cout) - cout)))


def kernel(node_features, edges, params):
    nf_pad = jnp.pad(node_features.astype(_F32), ((0, _NP - _N), (0, 0)))
    src = edges[:, 0].astype(jnp.int32)
    dst = edges[:, 1].astype(jnp.int32)
    padn = _EP - src.shape[0]
    fill = jnp.arange(padn, dtype=jnp.int32)
    srcp = jnp.concatenate([src, (fill * 1021) % _N])
    dstp = jnp.concatenate([dst, _N + (fill % _NT)])
    eil_a = jnp.stack([srcp.reshape(_NT, _NSA, _SL),
                       dstp.reshape(_NT, _NSA, _SL)], axis=2)
    eil_b = jnp.stack([srcp.reshape(2 * _NT, _NSB, _SL),
                       dstp.reshape(2 * _NT, _NSB, _SL)], axis=2)
    zrow = jnp.zeros((_NP, _CC), _F32)

    def agg(y):
        if y.shape[0] == 1:
            return _make_agg_split()(y, zrow, eil_b)
        return _make_agg_shared(y.shape[0])(y, eil_a)

    x = _colnorm(nf_pad)
    deg = agg(jnp.ones((1, _NP, _CC), _F32))
    dinv = _dinv_kernel(deg)

    def conv(act, w):
        return agg(_mm(act, dinv, _wpad(w)))

    p1 = params["conv1"]
    u = conv(x, p1["W"])
    a = _post(u, dinv, _pack_params(64, b=p1["b"]), "plain")

    for blk in params["blocks"]:
        cout = blk["conv1"]["W"].shape[1]
        u1 = conv(a, blk["conv1"]["W"])
        h = _post(u1, dinv,
                  _pack_params(cout, b=blk["conv1"]["b"],
                               g=blk["bn1"]["g"], bb=blk["bn1"]["b"]), "bn")
        u2 = conv(h, blk["conv2"]["W"])
        if "down" in blk:
            ud = conv(a, blk["down"]["conv"]["W"])
            pv = _pack_params(cout, b=blk["conv2"]["b"],
                              g=blk["bn2"]["g"], bb=blk["bn2"]["b"],
                              bd=blk["down"]["conv"]["b"],
                              gd=blk["down"]["bn"]["g"],
                              bbd=blk["down"]["bn"]["b"])
            a = _post(u2, dinv, pv, "down", extra=ud)
        else:
            pv = _pack_params(cout, b=blk["conv2"]["b"],
                              g=blk["bn2"]["g"], bb=blk["bn2"]["b"])
            a = _post(u2, dinv, pv, "res", extra=a)

    p2 = params["conv2"]
    u = conv(a, p2["W"])
    return _final(u, dinv, _pack_params(512, b=p2["b"]))[0]
